# Initial kernel scaffold; baseline (speedup 1.0000x reference)
import jax
import jax.numpy as jnp
from jax.experimental import pallas as pl


def kernel(x, geod_dist, W1, g1, b1, W2, g2, b2, W3, g3, b3, W4, g4, b4, W5, g5, b5, L1, g6, b6, L2, L2b, g7, b7, L3, L3b):
    raise NotImplementedError("write your pallas kernel here")


# speedup vs baseline: 5.6295x; 5.6295x over previous
import functools

import jax
import jax.numpy as jnp
from jax import lax
from jax.experimental import pallas as pl
from jax.experimental.pallas import tpu as pltpu, tpu_sc as plsc

F32 = jnp.float32
BF16 = jnp.bfloat16
HI = lax.Precision.HIGHEST
KNN = 20
BN_EPS = 1e-5
NB, NPTS = 4, 1024
BNROWS = NB * NPTS
ROWBLK = 512
GRID_R = BNROWS // ROWBLK
EROWS = 64
GRID_E = BNROWS // EROWS

NC, NS = 2, 16
NW = NC * NS
PPT = BNROWS // NW
PTS_PER_DMA = 4
NGROUPS = PPT // PTS_PER_DMA


def _lrelu(v):
    return jnp.where(v >= 0, v, 0.2 * v)


def _topk_body(g_ref, out_ref):
    b = pl.program_id(0)
    v = g_ref[0]
    iota = lax.broadcasted_iota(jnp.int32, (ROWBLK, NPTS), 1)
    cols = []
    for _ in range(KNN):
        m = jnp.max(v, axis=1, keepdims=True)
        cand = jnp.where(v == m, iota, NPTS)
        j = jnp.min(cand, axis=1, keepdims=True)
        cols.append(j)
        v = jnp.where(iota == j, -jnp.inf, v)
    out_ref[0] = jnp.concatenate(cols, axis=1) + b * NPTS


def _topk(geod):
    return pl.pallas_call(
        _topk_body,
        grid=(NB, NPTS // ROWBLK),
        in_specs=[pl.BlockSpec((1, ROWBLK, NPTS), lambda b, i: (b, i, 0))],
        out_specs=pl.BlockSpec((1, ROWBLK, KNN), lambda b, i: (b, i, 0)),
        out_shape=jax.ShapeDtypeStruct((NB, NPTS, KNN), jnp.int32),
    )(geod)


def _make_gather(CP):
    mesh = plsc.VectorSubcoreMesh(core_axis_name="c", subcore_axis_name="s")

    @functools.partial(
        pl.kernel,
        mesh=mesh,
        out_type=jax.ShapeDtypeStruct((BNROWS * KNN, CP), F32),
        scratch_types=[
            pltpu.VMEM((PPT * KNN,), jnp.int32),
            pltpu.VMEM((PTS_PER_DMA * KNN, CP), F32),
            pltpu.SemaphoreType.DMA,
        ],
    )
    def gr(idx_hbm, x_hbm, out_hbm, idx_v, gbuf, sem):
        wid = lax.axis_index("s") * NC + lax.axis_index("c")
        base = wid * (PPT * KNN)
        pltpu.sync_copy(idx_hbm.at[pl.ds(base, PPT * KNN)], idx_v)

        def group(g, carry):
            off = g * jnp.int32(PTS_PER_DMA * KNN)
            pltpu.async_copy(
                x_hbm.at[idx_v.at[pl.ds(off, PTS_PER_DMA * KNN)]],
                gbuf, sem).wait()
            pltpu.sync_copy(
                gbuf, out_hbm.at[pl.ds(base + off, PTS_PER_DMA * KNN)])
            return carry

        lax.fori_loop(jnp.int32(0), jnp.int32(NGROUPS), group, jnp.int32(0))

    return gr


_G_CACHE = {}


def _gather(idx_flat, xpad):
    CP = xpad.shape[1]
    if CP not in _G_CACHE:
        _G_CACHE[CP] = _make_gather(CP)
    return _G_CACHE[CP](idx_flat, xpad)


def _edge_body(g_ref, x_ref, w_ref, mx_ref, *, C, CP, O):
    g3 = g_ref[...].reshape(EROWS, KNN, CP)[:, :, :C]
    cen = x_ref[...][:, :C]
    d3 = g3 - cen[:, None, :]
    e3 = jnp.concatenate(
        [d3, jnp.broadcast_to(cen[:, None, :], (EROWS, KNN, C))], axis=2)
    e = e3.reshape(EROWS * KNN, 2 * C).astype(BF16)
    h = jnp.dot(e, w_ref[...], preferred_element_type=F32)
    mx_ref[...] = jnp.max(h.reshape(EROWS, KNN, O), axis=1)


def _edge(G, xpad, wt_bf, C):
    CP = xpad.shape[1]
    O = wt_bf.shape[1]
    return pl.pallas_call(
        functools.partial(_edge_body, C=C, CP=CP, O=O),
        grid=(GRID_E,),
        in_specs=[
            pl.BlockSpec((EROWS * KNN, CP), lambda i: (i, 0)),
            pl.BlockSpec((EROWS, CP), lambda i: (i, 0)),
            pl.BlockSpec((2 * C, O), lambda i: (0, 0)),
        ],
        out_specs=pl.BlockSpec((EROWS, O), lambda i: (i, 0)),
        out_shape=jax.ShapeDtypeStruct((BNROWS, O), F32),
    )(G, xpad, wt_bf)


def _apply_body(mx_ref, mu_ref, var_ref, g_ref, b_ref, x_ref, *, O, CPN):
    xv = (mx_ref[...] - mu_ref[...]) / jnp.sqrt(var_ref[...] + BN_EPS) \
        * g_ref[...] + b_ref[...]
    xv = _lrelu(xv)
    if CPN > O:
        xv = jnp.concatenate(
            [xv, jnp.zeros((ROWBLK, CPN - O), F32)], axis=1)
    x_ref[...] = xv


def _apply(mx, mu, var, gam, bet, CPN):
    O = mx.shape[1]
    return pl.pallas_call(
        functools.partial(_apply_body, O=O, CPN=CPN),
        grid=(GRID_R,),
        in_specs=[
            pl.BlockSpec((ROWBLK, O), lambda i: (i, 0)),
            pl.BlockSpec((1, O), lambda i: (0, 0)),
            pl.BlockSpec((1, O), lambda i: (0, 0)),
            pl.BlockSpec((1, O), lambda i: (0, 0)),
            pl.BlockSpec((1, O), lambda i: (0, 0)),
        ],
        out_specs=pl.BlockSpec((ROWBLK, CPN), lambda i: (i, 0)),
        out_shape=jax.ShapeDtypeStruct((BNROWS, CPN), F32),
    )(mx, mu, var, gam, bet)


def _head1_body(x1, x2, x3, x4, w5t, h5, maxv):
    i = pl.program_id(0)
    cat = jnp.concatenate([x1[...], x2[...], x3[...], x4[...]], axis=1)
    h = jnp.dot(cat.astype(BF16), w5t[...], preferred_element_type=F32)
    h5[...] = h
    emb = h.shape[1]
    mx = jnp.max(h, axis=0, keepdims=True)
    rowmask = lax.broadcasted_iota(jnp.int32, (8, emb), 0) == i // 2
    cur = jnp.where(rowmask, jnp.broadcast_to(mx, (8, emb)), -jnp.inf)

    @pl.when(i == 0)
    def _():
        maxv[...] = cur

    @pl.when(i > 0)
    def _():
        maxv[...] = jnp.maximum(maxv[...], cur)


def _head1(x1, x2, x3, x4, w5t_bf):
    emb = w5t_bf.shape[1]
    return pl.pallas_call(
        _head1_body,
        grid=(GRID_R,),
        in_specs=[
            pl.BlockSpec((ROWBLK, x1.shape[1]), lambda i: (i, 0)),
            pl.BlockSpec((ROWBLK, x2.shape[1]), lambda i: (i, 0)),
            pl.BlockSpec((ROWBLK, x3.shape[1]), lambda i: (i, 0)),
            pl.BlockSpec((ROWBLK, x4.shape[1]), lambda i: (i, 0)),
            pl.BlockSpec((512, emb), lambda i: (0, 0)),
        ],
        out_specs=[
            pl.BlockSpec((ROWBLK, emb), lambda i: (i, 0)),
            pl.BlockSpec((8, emb), lambda i: (0, 0)),
        ],
        out_shape=[
            jax.ShapeDtypeStruct((BNROWS, emb), F32),
            jax.ShapeDtypeStruct((8, emb), F32),
        ],
    )(x1, x2, x3, x4, w5t_bf)


def _head2_body(h5, mu, var, g5, b5, msum):
    i = pl.program_id(0)
    hn = _lrelu((h5[...] - mu[...]) / jnp.sqrt(var[...] + BN_EPS)
                * g5[...] + b5[...])
    emb = hn.shape[1]
    s = jnp.sum(hn, axis=0, keepdims=True)
    rowmask = lax.broadcasted_iota(jnp.int32, (8, emb), 0) == i // 2
    cur = jnp.where(rowmask, jnp.broadcast_to(s, (8, emb)), 0.0)

    @pl.when(i == 0)
    def _():
        msum[...] = cur

    @pl.when(i > 0)
    def _():
        msum[...] = msum[...] + cur


def _head2(h5, mu, var, g5, b5):
    emb = h5.shape[1]
    return pl.pallas_call(
        _head2_body,
        grid=(GRID_R,),
        in_specs=[
            pl.BlockSpec((ROWBLK, emb), lambda i: (i, 0)),
            pl.BlockSpec((1, emb), lambda i: (0, 0)),
            pl.BlockSpec((1, emb), lambda i: (0, 0)),
            pl.BlockSpec((1, emb), lambda i: (0, 0)),
            pl.BlockSpec((1, emb), lambda i: (0, 0)),
        ],
        out_specs=pl.BlockSpec((8, emb), lambda i: (0, 0)),
        out_shape=jax.ShapeDtypeStruct((8, emb), F32),
    )(h5, mu, var, g5, b5)


def _head3_body(maxv, msum, mu5, rs5, g5, b5, l1t, g6, b6, l2t, l2b, g7, b7,
                l3t, l3b, out):
    hm = _lrelu((maxv[0:NB, :] - mu5[...]) * rs5[...] * g5[...] + b5[...])
    mv = msum[0:NB, :]
    p = jnp.concatenate([hm, mv], axis=1)

    def bn0(z, g, b):
        mu = jnp.sum(z, axis=0, keepdims=True) * (1.0 / NB)
        var = jnp.sum(z * z, axis=0, keepdims=True) * (1.0 / NB) - mu * mu
        return (z - mu) * (lax.rsqrt(var + BN_EPS) * g) + b

    z = _lrelu(bn0(jnp.dot(p.astype(BF16), l1t[...],
                           preferred_element_type=F32), g6[...], b6[...]))
    z = _lrelu(bn0(jnp.dot(z.astype(BF16), l2t[...],
                           preferred_element_type=F32) + l2b[...],
                   g7[...], b7[...]))
    out[...] = jnp.dot(z.astype(BF16), l3t[...],
                       preferred_element_type=F32) + l3b[...]


def _head3(maxv, msum, mu5, rs5, g5, b5, l1t, g6, b6, l2t, l2b, g7, b7,
           l3t, l3b):
    args = (maxv, msum, mu5, rs5, g5, b5, l1t, g6, b6, l2t, l2b, g7, b7,
            l3t, l3b)
    return pl.pallas_call(
        _head3_body,
        in_specs=[pl.BlockSpec(a.shape, lambda *_, nd=a.ndim: (0,) * nd)
                  for a in args],
        out_specs=pl.BlockSpec((NB, 40), lambda: (0, 0)),
        out_shape=jax.ShapeDtypeStruct((NB, 40), F32),
    )(*args)


def kernel(x, geod_dist, W1, g1, b1, W2, g2, b2, W3, g3, b3, W4, g4, b4,
           W5, g5, b5, L1, g6, b6, L2, L2b, g7, b7, L3, L3b):
    r2 = lambda v: v.reshape(1, -1)

    xt = jnp.transpose(x, (0, 2, 1)).reshape(BNROWS, 3)
    xpad1 = jnp.pad(xt, ((0, 0), (0, 125)))

    gidx = _topk(geod_dist).reshape(BNROWS * KNN)

    def layer(xpad, C, CP, W, gam, bet, CPN):
        O = W.shape[0]
        del CP
        G = _gather(gidx, xpad)
        mx = _edge(G, xpad, W.T.astype(BF16), C)
        cen = xpad[:, :C]
        d = G[:, :C].reshape(BNROWS, KNN, C) - cen[:, None, :]
        e = jnp.concatenate(
            [d, jnp.broadcast_to(cen[:, None, :], (BNROWS, KNN, C))], axis=2)
        hx = jnp.dot(e.reshape(BNROWS * KNN, 2 * C).astype(BF16),
                     W.T.astype(BF16), preferred_element_type=F32)
        hb = jnp.transpose(hx.reshape(NB, NPTS, KNN, O), (0, 3, 1, 2))
        mu = r2(jnp.mean(hb, axis=(0, 2, 3)))
        var = r2(jnp.var(hb, axis=(0, 2, 3)))
        return _apply(mx, mu, var, r2(gam), r2(bet), CPN)

    xp2 = layer(xpad1, 3, 128, W1, g1, b1, 128)
    xp3 = layer(xp2, 64, 128, W2, g2, b2, 128)
    xp4 = layer(xp3, 64, 128, W3, g3, b3, 128)
    x4 = layer(xp4, 128, 128, W4, g4, b4, 256)

    x1 = xp2[:, :64]
    x2 = xp3[:, :64]
    x3 = xp4
    h5, maxv = _head1(x1, x2, x3, x4, W5.T.astype(BF16))
    cat = jnp.concatenate([x1, x2, x3, x4], axis=1)
    hx5 = jnp.dot(cat.astype(BF16), W5.T.astype(BF16),
                  preferred_element_type=F32)
    s1 = jnp.sum(hx5, axis=0)
    s2 = jnp.sum(hx5 * hx5, axis=0)
    mu5v = s1 / BNROWS
    rs5v = lax.rsqrt(s2 / BNROWS - mu5v * mu5v + BN_EPS)
    hn = _lrelu((hx5 - mu5v) * rs5v * g5.reshape(1, -1) + b5.reshape(1, -1))
    mv = jnp.mean(hn.reshape(NB, NPTS, -1), axis=1)
    mvpad = jnp.pad(mv, ((0, 4), (0, 0)))
    return _head3(maxv, mvpad, r2(mu5v), r2(rs5v), r2(g5), r2(b5),
                  L1.T.astype(BF16), r2(g6), r2(b6),
                  L2.T.astype(BF16), r2(L2b), r2(g7), r2(b7),
                  L3.T.astype(BF16), r2(L3b))
